# baseline (speedup 1.0000x reference)
import functools

import jax
import jax.numpy as jnp
from jax import lax
from jax.experimental import pallas as pl
from jax.experimental.pallas import tpu as pltpu
from jax.experimental.pallas import tpu_sc as plsc

_C = 4096
_L = 16
_NC, _NS = 2, 16
_NW = _NC * _NS
_TR = 8


def _sc_permute(x3, perm):
    n_tr = x3.shape[0]
    tr_per_w = n_tr // _NW
    mesh = plsc.VectorSubcoreMesh(core_axis_name="c", subcore_axis_name="s")

    @functools.partial(
        pl.kernel,
        out_type=jax.ShapeDtypeStruct((n_tr, _TR, _C), jnp.float32),
        mesh=mesh,
        compiler_params=pltpu.CompilerParams(
            needs_layout_passes=False, use_tc_tiling_on_sc=True
        ),
        scratch_types=[
            pltpu.VMEM((_C,), jnp.int32),
            pltpu.VMEM((_TR, _C), jnp.float32),
            pltpu.VMEM((_TR, _C), jnp.float32),
            pltpu.VMEM((_TR, _C // 2), jnp.float32),
            pltpu.VMEM((_TR, _C // 2), jnp.float32),
            pltpu.SemaphoreType.DMA,
            pltpu.SemaphoreType.DMA,
            pltpu.SemaphoreType.DMA,
            pltpu.SemaphoreType.DMA,
        ],
    )
    def k(x_hbm, perm_hbm, out_hbm, perm_v, in0, in1, oh0, oh1,
          si0, si1, so0, so1):
        wid = lax.axis_index("s") * _NC + lax.axis_index("c")
        tr0 = wid * tr_per_w
        pltpu.sync_copy(perm_hbm, perm_v)
        ins, sis = (in0, in1), (si0, si1)
        ohs, sos = (oh0, oh1), (so0, so1)

        def start_in(i, b):
            pltpu.async_copy(x_hbm.at[tr0 + i], ins[b], sis[b])

        def wait_in(b):
            pltpu.make_async_copy(x_hbm.at[0], ins[b], sis[b]).wait()

        def start_out(i, h):
            pltpu.async_copy(
                ohs[h],
                out_hbm.at[tr0 + i, :, pl.ds(h * (_C // 2), _C // 2)],
                sos[h])

        def wait_out(h):
            pltpu.make_async_copy(
                ohs[h],
                out_hbm.at[0, :, pl.ds(0, _C // 2)],
                sos[h]).wait()

        start_in(0, 0)
        start_in(1, 1)
        sr_splats = [jnp.full((_L,), sr, jnp.int32) for sr in range(_TR)]

        def loop_body(i2, carry):
            for b in range(2):
                i = i2 * 2 + b
                wait_in(b)
                in_v = ins[b]
                for h in range(2):
                    @pl.when(i2 > 0)
                    def _():
                        wait_out(h)

                    out_v = ohs[h]

                    def chunk_body(c, carry2, h=h, in_v=in_v, out_v=out_v):
                        col = c * _L
                        p = perm_v[pl.ds(h * (_C // 2) + col, _L)]
                        for sr in range(_TR):
                            vals = plsc.load_gather(in_v, [sr_splats[sr], p])
                            out_v[sr, pl.ds(col, _L)] = vals
                        return carry2

                    lax.fori_loop(0, _C // _L // 2, chunk_body, 0, unroll=2)
                    start_out(i, h)

                @pl.when(i + 2 < tr_per_w)
                def _():
                    start_in(i + 2, b)

            return carry

        lax.fori_loop(0, tr_per_w // 2, loop_body, 0)
        wait_out(0)
        wait_out(1)

    return k(x3, perm)


def kernel(x, perm):
    rows = x.shape[0] * x.shape[1]
    out = _sc_permute(x.reshape(rows // _TR, _TR, _C), perm)
    return out.reshape(x.shape)


# speedup vs baseline: 2.5452x; 2.5452x over previous
import functools

import jax
import jax.numpy as jnp
from jax import lax
from jax.experimental import pallas as pl
from jax.experimental.pallas import tpu as pltpu
from jax.experimental.pallas import tpu_sc as plsc

_C = 4096
_L = 16
_NC, _NS = 2, 16
_NW = _NC * _NS
_TR = 8
_NQ = 4
_QC = _C // _NQ


def _sc_permute(x3, perm):
    n_tr = x3.shape[0]
    tr_per_w = n_tr // _NW
    mesh = plsc.VectorSubcoreMesh(core_axis_name="c", subcore_axis_name="s")

    @functools.partial(
        pl.kernel,
        out_type=jax.ShapeDtypeStruct((n_tr, _TR, _C), jnp.float32),
        mesh=mesh,
        compiler_params=pltpu.CompilerParams(
            needs_layout_passes=False, use_tc_tiling_on_sc=True
        ),
        scratch_types=[
            pltpu.VMEM((_C + _L,), jnp.int32),
            pltpu.VMEM((_TR, _C), jnp.float32),
            pltpu.VMEM((_TR, _C), jnp.float32),
            pltpu.VMEM((_TR, _QC), jnp.float32),
            pltpu.VMEM((_TR, _QC), jnp.float32),
            pltpu.VMEM((_TR, _QC), jnp.float32),
            pltpu.VMEM((_TR, _QC), jnp.float32),
            pltpu.SemaphoreType.DMA,
            pltpu.SemaphoreType.DMA,
            pltpu.SemaphoreType.DMA,
            pltpu.SemaphoreType.DMA,
            pltpu.SemaphoreType.DMA,
            pltpu.SemaphoreType.DMA,
        ],
    )
    def k(x_hbm, perm_hbm, out_hbm, perm_v, in0, in1, oq0, oq1, oq2, oq3,
          si0, si1, sq0, sq1, sq2, sq3):
        wid = lax.axis_index("s") * _NC + lax.axis_index("c")
        tr0 = wid * tr_per_w
        pltpu.sync_copy(perm_hbm, perm_v.at[pl.ds(0, _C)])
        ins, sis = (in0, in1), (si0, si1)
        oqs, sqs = (oq0, oq1, oq2, oq3), (sq0, sq1, sq2, sq3)

        def start_in(i, b):
            pltpu.async_copy(x_hbm.at[tr0 + i], ins[b], sis[b])

        def wait_in(b):
            pltpu.make_async_copy(x_hbm.at[0], ins[b], sis[b]).wait()

        def start_out(i, q):
            pltpu.async_copy(
                oqs[q],
                out_hbm.at[tr0 + i, :, pl.ds(q * _QC, _QC)],
                sqs[q])

        def wait_out(q):
            pltpu.make_async_copy(
                oqs[q],
                out_hbm.at[0, :, pl.ds(0, _QC)],
                sqs[q]).wait()

        start_in(0, 0)
        start_in(1, 1)
        sr_splats = [jnp.full((_L,), sr, jnp.int32) for sr in range(_TR)]

        def loop_body(i2, carry):
            for b in range(2):
                i = i2 * 2 + b
                wait_in(b)
                in_v = ins[b]
                for q in range(_NQ):
                    @pl.when(i > 0)
                    def _():
                        wait_out(q)

                    out_v = oqs[q]

                    def chunk_body(c, p, q=q, in_v=in_v, out_v=out_v):
                        col = c * _L
                        p_next = perm_v[pl.ds(q * _QC + col + _L, _L)]
                        vals = [plsc.load_gather(in_v, [sr_splats[sr], p])
                                for sr in range(_TR)]
                        for sr in range(_TR):
                            out_v[sr, pl.ds(col, _L)] = vals[sr]
                        return p_next

                    p0 = perm_v[pl.ds(q * _QC, _L)]
                    lax.fori_loop(0, _QC // _L, chunk_body, p0, unroll=2)
                    start_out(i, q)

                @pl.when(i + 2 < tr_per_w)
                def _():
                    start_in(i + 2, b)

            return carry

        lax.fori_loop(0, tr_per_w // 2, loop_body, 0)
        for q in range(_NQ):
            wait_out(q)

    return k(x3, perm)


def kernel(x, perm):
    rows = x.shape[0] * x.shape[1]
    out = _sc_permute(x.reshape(rows // _TR, _TR, _C), perm)
    return out.reshape(x.shape)
